# baseline (speedup 1.0000x reference)
import functools

import jax
import jax.numpy as jnp
from jax import lax
from jax.experimental import pallas as pl
from jax.experimental.pallas import tpu as pltpu
from jax.experimental.pallas import tpu_sc as plsc

_K = 128
_N = 16 * 128 * 128
_CHUNK = 16


def _sc_gather_affine(x_flat, et_flat, mul_flat, bias_flat):
    n_elems = x_flat.shape[0]
    info = plsc.get_sparse_core_info()
    nc, ns = info.num_cores, info.num_subcores
    nw = nc * ns
    per_w = n_elems // nw
    table_n = mul_flat.shape[0]

    mesh = plsc.VectorSubcoreMesh(core_axis_name="c", subcore_axis_name="s")

    @functools.partial(
        pl.kernel,
        out_type=jax.ShapeDtypeStruct((n_elems,), jnp.float32),
        mesh=mesh,
        compiler_params=pltpu.CompilerParams(needs_layout_passes=False),
        scratch_types=[
            pltpu.VMEM((per_w,), jnp.int32),
            pltpu.VMEM((per_w,), jnp.float32),
            pltpu.VMEM((per_w,), jnp.float32),
            pltpu.VMEM((table_n,), jnp.float32),
            pltpu.VMEM((table_n,), jnp.float32),
            pltpu.SemaphoreType.DMA,
            pltpu.SemaphoreType.DMA,
            pltpu.SemaphoreType.DMA,
            pltpu.SemaphoreType.DMA,
        ],
    )
    def sck(x_hbm, et_hbm, mul_hbm, bias_hbm, y_hbm,
            idx_v, x_v, y_v, mul_v, bias_v, s0, s1, s2, s3):
        wid = lax.axis_index("s") * nc + lax.axis_index("c")
        base = wid * per_w
        c0 = pltpu.async_copy(mul_hbm, mul_v, s0)
        c1 = pltpu.async_copy(bias_hbm, bias_v, s1)
        c2 = pltpu.async_copy(et_hbm.at[pl.ds(base, per_w)], idx_v, s2)
        c3 = pltpu.async_copy(x_hbm.at[pl.ds(base, per_w)], x_v, s3)
        c0.wait()
        c1.wait()
        c2.wait()
        c3.wait()

        @plsc.parallel_loop(0, per_w, _CHUNK, unroll=8)
        def body(o):
            idx = idx_v[pl.ds(o, _CHUNK)]
            xv = x_v[pl.ds(o, _CHUNK)]
            m = plsc.load_gather(mul_v, [idx])
            b = plsc.load_gather(bias_v, [idx])
            y_v[pl.ds(o, _CHUNK)] = m * xv + b

        pltpu.sync_copy(y_v, y_hbm.at[pl.ds(base, per_w)])

    return sck(x_flat, et_flat, mul_flat, bias_flat)


def _gaussian_math(y_ref, means_ref, stds_ref, o_ref):
    mean = means_ref[...].reshape(1, 1, _K)
    std = jnp.abs(stds_ref[...]).reshape(1, 1, _K) + 1e-05
    inv = 1.0 / std
    coef = inv * (1.0 / ((2.0 * 3.14159) ** 0.5))
    y = y_ref[...][:, :, None]
    t = (y - mean) * inv
    o_ref[...] = jnp.exp(t * t * -0.5) * coef


def _tc_body_first(y_ref, means_ref, stds_ref, o_ref):
    _gaussian_math(y_ref, means_ref, stds_ref, o_ref)


def _tc_body_second(y_ref, means_ref, stds_ref, buf_ref, o_ref):
    del buf_ref
    _gaussian_math(y_ref, means_ref, stds_ref, o_ref)


_R = 256
_ROWS = _N // 128


def _tc_gaussian_half(y_half, means, stds, out_prev, block_off):
    rows_half = y_half.shape[0] // 128
    grid = rows_half // _R
    y2 = y_half.reshape(rows_half, 128)
    common = dict(
        grid=(grid,),
        out_specs=pl.BlockSpec(
            (_R, 128, _K), lambda i: (i + block_off, 0, 0)
        ),
        out_shape=jax.ShapeDtypeStruct((_ROWS, 128, _K), jnp.float32),
        compiler_params=pltpu.CompilerParams(
            dimension_semantics=("arbitrary",),
        ),
    )
    in_specs = [
        pl.BlockSpec((_R, 128), lambda i: (i, 0)),
        pl.BlockSpec((1, _K), lambda i: (0, 0)),
        pl.BlockSpec((1, _K), lambda i: (0, 0)),
    ]
    if out_prev is None:
        return pl.pallas_call(
            _tc_body_first, in_specs=in_specs, **common
        )(y2, means, stds)
    in_specs.append(pl.BlockSpec(memory_space=pl.ANY))
    return pl.pallas_call(
        _tc_body_second,
        in_specs=in_specs,
        input_output_aliases={3: 0},
        **common,
    )(y2, means, stds, out_prev)


def kernel(x, edge_type, means, stds, mul_table, bias_table):
    b, n, m = x.shape
    x_flat = x.reshape(_N)
    et_flat = edge_type.reshape(_N).astype(jnp.int32)
    mul_flat = mul_table.reshape(-1)
    bias_flat = bias_table.reshape(-1)
    half = _N // 2
    y_a = _sc_gather_affine(x_flat[:half], et_flat[:half], mul_flat, bias_flat)
    y_b = _sc_gather_affine(x_flat[half:], et_flat[half:], mul_flat, bias_flat)
    out_a = _tc_gaussian_half(y_a, means, stds, None, 0)
    out = _tc_gaussian_half(y_b, means, stds, out_a, _ROWS // 2 // _R)
    return out.reshape(b, n, m, _K)


# speedup vs baseline: 1.0834x; 1.0834x over previous
import functools

import jax
import jax.numpy as jnp
from jax import lax
from jax.experimental import pallas as pl
from jax.experimental.pallas import tpu as pltpu
from jax.experimental.pallas import tpu_sc as plsc

_K = 128
_N = 16 * 128 * 128
_CHUNK = 16


def _sc_gather_affine(x2, et2, mul_flat, bias_flat):
    rows = x2.shape[0]
    info = plsc.get_sparse_core_info()
    nc, ns = info.num_cores, info.num_subcores
    nw = nc * ns
    rows_w = rows // nw
    table_n = mul_flat.shape[0]

    mesh = plsc.VectorSubcoreMesh(core_axis_name="c", subcore_axis_name="s")

    @functools.partial(
        pl.kernel,
        out_type=jax.ShapeDtypeStruct((rows, 128), jnp.float32),
        mesh=mesh,
        compiler_params=pltpu.CompilerParams(needs_layout_passes=False),
        scratch_types=[
            pltpu.VMEM((rows_w, 128), jnp.int32),
            pltpu.VMEM((rows_w, 128), jnp.float32),
            pltpu.VMEM((rows_w, 128), jnp.float32),
            pltpu.VMEM((table_n,), jnp.float32),
            pltpu.VMEM((table_n,), jnp.float32),
            pltpu.SemaphoreType.DMA,
            pltpu.SemaphoreType.DMA,
            pltpu.SemaphoreType.DMA,
            pltpu.SemaphoreType.DMA,
        ],
    )
    def sck(x_hbm, et_hbm, mul_hbm, bias_hbm, y_hbm,
            idx_v, x_v, y_v, mul_v, bias_v, s0, s1, s2, s3):
        wid = lax.axis_index("s") * nc + lax.axis_index("c")
        r0 = wid * rows_w
        c0 = pltpu.async_copy(mul_hbm, mul_v, s0)
        c1 = pltpu.async_copy(bias_hbm, bias_v, s1)
        c2 = pltpu.async_copy(et_hbm.at[pl.ds(r0, rows_w)], idx_v, s2)
        c3 = pltpu.async_copy(x_hbm.at[pl.ds(r0, rows_w)], x_v, s3)
        c0.wait()
        c1.wait()
        c2.wait()
        c3.wait()

        @plsc.parallel_loop(0, rows_w, 1, unroll=2)
        def body(rr):
            for cc in range(0, 128, _CHUNK):
                idx = idx_v[rr, pl.ds(cc, _CHUNK)]
                xv = x_v[rr, pl.ds(cc, _CHUNK)]
                m = plsc.load_gather(mul_v, [idx])
                b = plsc.load_gather(bias_v, [idx])
                y_v[rr, pl.ds(cc, _CHUNK)] = m * xv + b

        pltpu.sync_copy(y_v, y_hbm.at[pl.ds(r0, rows_w)])

    return sck(x2, et2, mul_flat, bias_flat)


def _tc_gaussian_body(y_ref, means_ref, stds_ref, o_ref):
    mean = means_ref[...].reshape(1, 1, _K)
    std = jnp.abs(stds_ref[...]).reshape(1, 1, _K) + 1e-05
    inv = 1.0 / std
    coef = inv * (1.0 / ((2.0 * 3.14159) ** 0.5))
    y = y_ref[...][:, :, None]
    t = (y - mean) * inv
    o_ref[...] = jnp.exp(t * t * -0.5) * coef


def _tc_gaussian(y2, means, stds):
    rows = y2.shape[0]
    r = 256
    grid = rows // r
    return pl.pallas_call(
        _tc_gaussian_body,
        grid=(grid,),
        in_specs=[
            pl.BlockSpec((r, 128), lambda i: (i, 0)),
            pl.BlockSpec((1, _K), lambda i: (0, 0)),
            pl.BlockSpec((1, _K), lambda i: (0, 0)),
        ],
        out_specs=pl.BlockSpec((r, 128, _K), lambda i: (i, 0, 0)),
        out_shape=jax.ShapeDtypeStruct((rows, 128, _K), jnp.float32),
        compiler_params=pltpu.CompilerParams(
            dimension_semantics=("arbitrary",),
        ),
    )(y2, means, stds)


def kernel(x, edge_type, means, stds, mul_table, bias_table):
    b, n, m = x.shape
    rows = _N // 128
    x2 = x.reshape(rows, 128)
    et2 = edge_type.reshape(rows, 128).astype(jnp.int32)
    y2 = _sc_gather_affine(
        x2, et2, mul_table.reshape(-1), bias_table.reshape(-1)
    )
    out = _tc_gaussian(y2, means, stds)
    return out.reshape(b, n, m, _K)


# speedup vs baseline: 1.0935x; 1.0093x over previous
import functools

import jax
import jax.numpy as jnp
from jax import lax
from jax.experimental import pallas as pl
from jax.experimental.pallas import tpu as pltpu
from jax.experimental.pallas import tpu_sc as plsc

_K = 128
_N = 16 * 128 * 128
_CHUNK = 16


def _sc_gather_affine(x_flat, et_flat, mul_flat, bias_flat):
    info = plsc.get_sparse_core_info()
    nc, ns = info.num_cores, info.num_subcores
    nw = nc * ns
    per_w = _N // nw
    table_n = mul_flat.shape[0]

    mesh = plsc.VectorSubcoreMesh(core_axis_name="c", subcore_axis_name="s")

    @functools.partial(
        pl.kernel,
        out_type=jax.ShapeDtypeStruct((_N,), jnp.float32),
        mesh=mesh,
        compiler_params=pltpu.CompilerParams(needs_layout_passes=False),
        scratch_types=[
            pltpu.VMEM((per_w,), jnp.int32),
            pltpu.VMEM((per_w,), jnp.float32),
            pltpu.VMEM((per_w,), jnp.float32),
            pltpu.VMEM((table_n,), jnp.float32),
            pltpu.VMEM((table_n,), jnp.float32),
            pltpu.SemaphoreType.DMA,
            pltpu.SemaphoreType.DMA,
            pltpu.SemaphoreType.DMA,
            pltpu.SemaphoreType.DMA,
        ],
    )
    def sck(x_hbm, et_hbm, mul_hbm, bias_hbm, y_hbm,
            idx_v, x_v, y_v, mul_v, bias_v, s0, s1, s2, s3):
        wid = lax.axis_index("s") * nc + lax.axis_index("c")
        base = wid * per_w
        c0 = pltpu.async_copy(mul_hbm, mul_v, s0)
        c1 = pltpu.async_copy(bias_hbm, bias_v, s1)
        c2 = pltpu.async_copy(et_hbm.at[pl.ds(base, per_w)], idx_v, s2)
        c3 = pltpu.async_copy(x_hbm.at[pl.ds(base, per_w)], x_v, s3)
        c0.wait()
        c1.wait()
        c2.wait()
        c3.wait()

        @plsc.parallel_loop(0, per_w, _CHUNK, unroll=16)
        def body(o):
            idx = idx_v[pl.ds(o, _CHUNK)]
            xv = x_v[pl.ds(o, _CHUNK)]
            m = plsc.load_gather(mul_v, [idx])
            b = plsc.load_gather(bias_v, [idx])
            y_v[pl.ds(o, _CHUNK)] = m * xv + b

        pltpu.sync_copy(y_v, y_hbm.at[pl.ds(base, per_w)])

    return sck(x_flat, et_flat, mul_flat, bias_flat)


def _tc_gaussian_body(y_ref, means_ref, stds_ref, o_ref):
    mean = means_ref[...].reshape(1, 1, _K)
    std = jnp.abs(stds_ref[...]).reshape(1, 1, _K) + 1e-05
    inv = 1.0 / std
    coef = inv * (1.0 / ((2.0 * 3.14159) ** 0.5))
    y = y_ref[...][:, :, None]
    t = (y - mean) * inv
    o_ref[...] = jnp.exp(t * t * -0.5) * coef


def _tc_gaussian(y_flat, means, stds):
    rows = _N // 128
    r = 256
    grid = rows // r
    return pl.pallas_call(
        _tc_gaussian_body,
        grid=(grid,),
        in_specs=[
            pl.BlockSpec((r, 128), lambda i: (i, 0)),
            pl.BlockSpec((1, _K), lambda i: (0, 0)),
            pl.BlockSpec((1, _K), lambda i: (0, 0)),
        ],
        out_specs=pl.BlockSpec((r, 128, _K), lambda i: (i, 0, 0)),
        out_shape=jax.ShapeDtypeStruct((rows, 128, _K), jnp.float32),
        compiler_params=pltpu.CompilerParams(
            dimension_semantics=("arbitrary",),
        ),
    )(y_flat.reshape(rows, 128), means, stds)


def kernel(x, edge_type, means, stds, mul_table, bias_table):
    b, n, m = x.shape
    x_flat = x.reshape(_N)
    et_flat = edge_type.reshape(_N).astype(jnp.int32)
    y_flat = _sc_gather_affine(
        x_flat, et_flat, mul_table.reshape(-1), bias_table.reshape(-1)
    )
    out = _tc_gaussian(y_flat, means, stds)
    return out.reshape(b, n, m, _K)


# speedup vs baseline: 1.0983x; 1.0044x over previous
import functools

import jax
import jax.numpy as jnp
from jax import lax
from jax.experimental import pallas as pl
from jax.experimental.pallas import tpu as pltpu
from jax.experimental.pallas import tpu_sc as plsc

_K = 128
_N = 16 * 128 * 128
_CHUNK = 16


def _sc_gather_affine(x_flat, et_flat, mul_flat, bias_flat):
    info = plsc.get_sparse_core_info()
    nc, ns = info.num_cores, info.num_subcores
    nw = nc * ns
    per_w = _N // nw
    table_n = mul_flat.shape[0]

    mesh = plsc.VectorSubcoreMesh(core_axis_name="c", subcore_axis_name="s")

    @functools.partial(
        pl.kernel,
        out_type=jax.ShapeDtypeStruct((_N,), jnp.float32),
        mesh=mesh,
        compiler_params=pltpu.CompilerParams(needs_layout_passes=False),
        scratch_types=[
            pltpu.VMEM((per_w,), jnp.int32),
            pltpu.VMEM((per_w,), jnp.float32),
            pltpu.VMEM((per_w,), jnp.float32),
            pltpu.VMEM((table_n,), jnp.float32),
            pltpu.VMEM((table_n,), jnp.float32),
            pltpu.SemaphoreType.DMA,
            pltpu.SemaphoreType.DMA,
            pltpu.SemaphoreType.DMA,
            pltpu.SemaphoreType.DMA,
        ],
    )
    def sck(x_hbm, et_hbm, mul_hbm, bias_hbm, y_hbm,
            idx_v, x_v, y_v, mul_v, bias_v, s0, s1, s2, s3):
        wid = lax.axis_index("s") * nc + lax.axis_index("c")
        base = wid * per_w
        c0 = pltpu.async_copy(mul_hbm, mul_v, s0)
        c1 = pltpu.async_copy(bias_hbm, bias_v, s1)
        c2 = pltpu.async_copy(et_hbm.at[pl.ds(base, per_w)], idx_v, s2)
        c3 = pltpu.async_copy(x_hbm.at[pl.ds(base, per_w)], x_v, s3)
        c0.wait()
        c1.wait()
        c2.wait()
        c3.wait()

        @plsc.parallel_loop(0, per_w, _CHUNK, unroll=8)
        def body(o):
            idx = idx_v[pl.ds(o, _CHUNK)]
            xv = x_v[pl.ds(o, _CHUNK)]
            m = plsc.load_gather(mul_v, [idx])
            b = plsc.load_gather(bias_v, [idx])
            y_v[pl.ds(o, _CHUNK)] = m * xv + b

        pltpu.sync_copy(y_v, y_hbm.at[pl.ds(base, per_w)])

    return sck(x_flat, et_flat, mul_flat, bias_flat)


def _tc_gaussian_body(y_ref, means_ref, stds_ref, o_ref):
    mean = means_ref[...].reshape(1, 1, _K)
    std = jnp.abs(stds_ref[...]).reshape(1, 1, _K) + 1e-05
    inv = 1.0 / std
    coef = inv * (1.0 / ((2.0 * 3.14159) ** 0.5))
    c2 = inv * inv * -0.5
    y = y_ref[...][:, :, None]
    d = y - mean
    o_ref[...] = jnp.exp(d * d * c2) * coef


def _tc_gaussian(y_flat, means, stds):
    rows = _N // 128
    r = 256
    grid = rows // r
    return pl.pallas_call(
        _tc_gaussian_body,
        grid=(grid,),
        in_specs=[
            pl.BlockSpec((r, 128), lambda i: (i, 0)),
            pl.BlockSpec((1, _K), lambda i: (0, 0)),
            pl.BlockSpec((1, _K), lambda i: (0, 0)),
        ],
        out_specs=pl.BlockSpec((r, 128, _K), lambda i: (i, 0, 0)),
        out_shape=jax.ShapeDtypeStruct((rows, 128, _K), jnp.float32),
        compiler_params=pltpu.CompilerParams(
            dimension_semantics=("arbitrary",),
        ),
    )(y_flat.reshape(rows, 128), means, stds)


def kernel(x, edge_type, means, stds, mul_table, bias_table):
    b, n, m = x.shape
    x_flat = x.reshape(_N)
    et_flat = edge_type.reshape(_N).astype(jnp.int32)
    y_flat = _sc_gather_affine(
        x_flat, et_flat, mul_table.reshape(-1), bias_table.reshape(-1)
    )
    out = _tc_gaussian(y_flat, means, stds)
    return out.reshape(b, n, m, _K)
